# baseline (speedup 1.0000x reference)
import functools

import jax
import jax.numpy as jnp
from jax import lax
from jax.experimental import pallas as pl
from jax.experimental.pallas import tpu as pltpu
from jax.experimental.pallas import tpu_sc as plsc

H = 64
M = 4
T = 24
V = 64
GP = 128
EMPTY = 64
NW = 32
L = 16


def _dot_t(a, b):
    return lax.dot_general(a, b, (((1,), (1,)), ((), ())),
                           preferred_element_type=jnp.float32)


def _dot(a, b):
    return lax.dot_general(a, b, (((1,), (0,)), ((), ())),
                           preferred_element_type=jnp.float32)


def _tables(embed, wg_W1, wg_b1, wg_W2, wg_b2, eg_W1, eg_b1, eg_W2, eg_b2):
    emb = embed[:V, :]
    h = jax.nn.relu(emb @ wg_W1.T + wg_b1)
    w = jax.nn.sigmoid(h @ wg_W2.T + wg_b2)
    v_voc = emb * w
    slot_vals = jnp.concatenate(
        [v_voc, jnp.zeros((1, H), jnp.float32)], axis=0)
    inp = jnp.concatenate(
        [jnp.broadcast_to(emb[:, None, :], (V, V + 1, H)),
         jnp.broadcast_to(slot_vals[None, :, :], (V, V + 1, H))],
        axis=-1)
    eh = jax.nn.relu(inp @ eg_W1.T + eg_b1)
    g = (eh @ eg_W2.T + eg_b2)[..., 0]
    g_pad = jnp.concatenate(
        [g, jnp.zeros((V, GP - (V + 1)), jnp.float32)], axis=1)
    return g_pad, v_voc


def _sc_scan_factory(B):
    rpw = B // NW
    ng = rpw // L
    mesh = plsc.VectorSubcoreMesh(core_axis_name="c", subcore_axis_name="s")
    f32 = jnp.float32

    @functools.partial(
        pl.kernel, mesh=mesh,
        out_type=(pltpu.HBM((B, H), f32),
                  pltpu.HBM((B, 2 * H), f32)),
        compiler_params=pltpu.CompilerParams(needs_layout_passes=False),
        scratch_types=[
            pltpu.VMEM((rpw * T,), jnp.int32),
            pltpu.VMEM((V * GP,), f32),
            pltpu.VMEM((rpw,), jnp.int32),
            pltpu.VMEM((rpw, H), f32),
            pltpu.VMEM((128, 2 * H), f32),
            pltpu.VMEM((128, 2 * H), f32),
            pltpu.SemaphoreType.DMA,
            pltpu.SemaphoreType.DMA,
        ],
    )
    def sc_scan(seqs_hbm, q_hbm, g_hbm, embed_hbm, cnt_out, qemb_out,
                seqs_v, g_v, qidx_v, cnt_v, qe0, qe1, sem, sem2):
        wid = lax.axis_index("s") * 2 + lax.axis_index("c")
        base = wid * rpw
        pltpu.sync_copy(q_hbm.at[pl.ds(base, rpw)], qidx_v)
        pltpu.sync_copy(g_hbm, g_v)
        pltpu.sync_copy(seqs_hbm.at[pl.ds(base * T, rpw * T)], seqs_v)
        zeros16 = jnp.zeros((L,), f32)
        lane = lax.iota(jnp.int32, L)
        ones16 = jnp.ones((L,), f32)

        NI = 4
        empty = jnp.full((L,), EMPTY, jnp.int32)

        def group(gj, carry):
            gi0 = gj * NI
            for i in range(NI):
                for j in range(L):
                    for c in range(H // L):
                        cnt_v[(gi0 + i) * L + j, pl.ds(c * L, L)] = zeros16
            st = [(empty, empty, empty, empty) for _ in range(NI)]
            offs = [(gi0 + i) * (L * T) + lane * T for i in range(NI)]
            for t in range(T - 1):
                curs = [plsc.load_gather(seqs_v, [offs[i] + t])
                        for i in range(NI)]
                for i in range(NI):
                    gbase = curs[i] * GP
                    slots = st[i]
                    logits = [plsc.load_gather(g_v, [gbase + s])
                              for s in slots]
                    best = logits[0]
                    bi = jnp.zeros((L,), jnp.int32)
                    for m in range(1, M):
                        win = logits[m] > best
                        best = jnp.where(win, logits[m], best)
                        bi = jnp.where(win, jnp.full((L,), m, jnp.int32), bi)
                    st[i] = tuple(
                        jnp.where(bi == m, curs[i], slots[m])
                        for m in range(M))
            for i in range(NI):
                gl = (gi0 + i) * L + lane
                for m in range(M):
                    plsc.addupdate_scatter(cnt_v, [gl, st[i][m]], ones16,
                                           mask=st[i][m] < EMPTY)
            return carry

        qbufs = (qe0, qe1)
        gpc = 128 // L
        pend = []
        outq = [None, None]
        for k in range(rpw // 128):
            b = qbufs[k % 2]
            if outq[k % 2] is not None:
                outq[k % 2].wait()
            gat = pltpu.async_copy(
                embed_hbm.at[qidx_v.at[pl.ds(k * 128, 128)]], b, sem)
            lax.fori_loop(k * gpc // NI, (k + 1) * gpc // NI, group, 0)
            gat.wait()
            outq[k % 2] = pltpu.async_copy(
                b, qemb_out.at[pl.ds(base + k * 128, 128)], sem2)
            pend.append(pltpu.async_copy(
                cnt_v.at[pl.ds(k * 128, 128)],
                cnt_out.at[pl.ds(base + k * 128, 128)], sem2))
        outq[0].wait()
        outq[1].wait()
        for c in pend:
            c.wait()

    return sc_scan


def _head_body(count, qemb, vvoc, rh_W1, rh_b1r, rh_W2, rh_b2r, out):
    mem_summary = _dot(count[...], vvoc[...]) * 0.25
    w1 = rh_W1[...]
    rh = jnp.maximum(
        _dot_t(qemb[...][:, :H], w1[:, :H]) + _dot_t(mem_summary, w1[:, H:])
        + rh_b1r[...], 0.0)
    out[...] = _dot_t(rh, rh_W2[...]) + rh_b2r[...]


def kernel(seqs, query_tok, embed, wg_W1, wg_b1, wg_W2, wg_b2,
           eg_W1, eg_b1, eg_W2, eg_b2, rh_W1, rh_b1, rh_W2, rh_b2):
    B = seqs.shape[0]
    seqs = seqs.astype(jnp.int32)
    f32 = jnp.float32

    g_pair, v_voc = _tables(embed, wg_W1, wg_b1, wg_W2, wg_b2,
                            eg_W1, eg_b1, eg_W2, eg_b2)

    embed_pad = jnp.concatenate(
        [embed, jnp.zeros(embed.shape, jnp.float32)], axis=1)
    count, q_emb = _sc_scan_factory(B)(
        seqs.reshape(-1), query_tok.astype(jnp.int32), g_pair.reshape(-1),
        embed_pad)

    TILE = 2048
    fullg = lambda s: pl.BlockSpec(s, lambda i: tuple(0 for _ in s))
    logits = pl.pallas_call(
        _head_body,
        grid=(B // TILE,),
        out_shape=jax.ShapeDtypeStruct((B, V), f32),
        in_specs=[pl.BlockSpec((TILE, H), lambda i: (i, 0)),
                  pl.BlockSpec((TILE, 2 * H), lambda i: (i, 0)),
                  fullg((V, H)), fullg((H, 2 * H)), fullg((1, H)),
                  fullg((V, H)), fullg((1, V))],
        out_specs=pl.BlockSpec((TILE, V), lambda i: (i, 0)),
    )(count, q_emb, v_voc, rh_W1, rh_b1.reshape(1, H), rh_W2,
      rh_b2.reshape(1, V))
    return logits


# speedup vs baseline: 1.0233x; 1.0233x over previous
import functools

import jax
import jax.numpy as jnp
from jax import lax
from jax.experimental import pallas as pl
from jax.experimental.pallas import tpu as pltpu
from jax.experimental.pallas import tpu_sc as plsc

H = 64
M = 4
T = 24
V = 64
GP = 128
EMPTY = 64
NW = 32
L = 16


def _dot_t(a, b):
    return lax.dot_general(a, b, (((1,), (1,)), ((), ())),
                           preferred_element_type=jnp.float32)


def _dot(a, b):
    return lax.dot_general(a, b, (((1,), (0,)), ((), ())),
                           preferred_element_type=jnp.float32)


def _tables(embed, wg_W1, wg_b1, wg_W2, wg_b2, eg_W1, eg_b1, eg_W2, eg_b2):
    emb = embed[:V, :]
    h = jax.nn.relu(emb @ wg_W1.T + wg_b1)
    w = jax.nn.sigmoid(h @ wg_W2.T + wg_b2)
    v_voc = emb * w
    slot_vals = jnp.concatenate(
        [v_voc, jnp.zeros((1, H), jnp.float32)], axis=0)
    inp = jnp.concatenate(
        [jnp.broadcast_to(emb[:, None, :], (V, V + 1, H)),
         jnp.broadcast_to(slot_vals[None, :, :], (V, V + 1, H))],
        axis=-1)
    eh = jax.nn.relu(inp @ eg_W1.T + eg_b1)
    g = (eh @ eg_W2.T + eg_b2)[..., 0]
    g_pad = jnp.concatenate(
        [g, jnp.zeros((V, GP - (V + 1)), jnp.float32)], axis=1)
    return g_pad, v_voc


def _sc_scan_factory(B):
    rpw = B // NW
    ng = rpw // L
    mesh = plsc.VectorSubcoreMesh(core_axis_name="c", subcore_axis_name="s")
    f32 = jnp.float32

    @functools.partial(
        pl.kernel, mesh=mesh,
        out_type=(pltpu.HBM((B, H), f32),
                  pltpu.HBM((B, 2 * H), f32)),
        compiler_params=pltpu.CompilerParams(needs_layout_passes=False),
        scratch_types=[
            pltpu.VMEM((rpw * T,), jnp.int32),
            pltpu.VMEM((V * GP,), f32),
            pltpu.VMEM((rpw,), jnp.int32),
            pltpu.VMEM((rpw, H), f32),
            pltpu.VMEM((128, 2 * H), f32),
            pltpu.VMEM((128, 2 * H), f32),
            pltpu.SemaphoreType.DMA,
            pltpu.SemaphoreType.DMA,
        ],
    )
    def sc_scan(seqs_hbm, q_hbm, g_hbm, embed_hbm, cnt_out, qemb_out,
                seqs_v, g_v, qidx_v, cnt_v, qe0, qe1, sem, sem2):
        wid = lax.axis_index("s") * 2 + lax.axis_index("c")
        base = wid * rpw
        pltpu.sync_copy(q_hbm.at[pl.ds(base, rpw)], qidx_v)
        pltpu.sync_copy(g_hbm, g_v)
        pltpu.sync_copy(seqs_hbm.at[pl.ds(base * T, rpw * T)], seqs_v)
        zeros16 = jnp.zeros((L,), f32)
        lane = lax.iota(jnp.int32, L)
        ones16 = jnp.ones((L,), f32)

        NI = 2
        empty = jnp.full((L,), EMPTY, jnp.int32)

        def group(gj, carry):
            gi0 = gj * NI
            for i in range(NI):
                for j in range(L):
                    for c in range(H // L):
                        cnt_v[(gi0 + i) * L + j, pl.ds(c * L, L)] = zeros16
            st = [(empty, empty, empty, empty) for _ in range(NI)]
            offs = [(gi0 + i) * (L * T) + lane * T for i in range(NI)]
            for t in range(T - 1):
                curs = [plsc.load_gather(seqs_v, [offs[i] + t])
                        for i in range(NI)]
                for i in range(NI):
                    gbase = curs[i] * GP
                    slots = st[i]
                    logits = [plsc.load_gather(g_v, [gbase + s])
                              for s in slots]
                    best = logits[0]
                    bi = jnp.zeros((L,), jnp.int32)
                    for m in range(1, M):
                        win = logits[m] > best
                        best = jnp.where(win, logits[m], best)
                        bi = jnp.where(win, jnp.full((L,), m, jnp.int32), bi)
                    st[i] = tuple(
                        jnp.where(bi == m, curs[i], slots[m])
                        for m in range(M))
            for i in range(NI):
                gl = (gi0 + i) * L + lane
                for m in range(M):
                    plsc.addupdate_scatter(cnt_v, [gl, st[i][m]], ones16,
                                           mask=st[i][m] < EMPTY)
            return carry

        qbufs = (qe0, qe1)
        gpc = 128 // L
        pend = []
        outq = [None, None]
        for k in range(rpw // 128):
            b = qbufs[k % 2]
            if outq[k % 2] is not None:
                outq[k % 2].wait()
            gat = pltpu.async_copy(
                embed_hbm.at[qidx_v.at[pl.ds(k * 128, 128)]], b, sem)
            lax.fori_loop(k * gpc // NI, (k + 1) * gpc // NI, group, 0)
            gat.wait()
            outq[k % 2] = pltpu.async_copy(
                b, qemb_out.at[pl.ds(base + k * 128, 128)], sem2)
            pend.append(pltpu.async_copy(
                cnt_v.at[pl.ds(k * 128, 128)],
                cnt_out.at[pl.ds(base + k * 128, 128)], sem2))
        outq[0].wait()
        outq[1].wait()
        for c in pend:
            c.wait()

    return sc_scan


def _head_body(count, qemb, vvoc, rh_W1, rh_b1r, rh_W2, rh_b2r, out):
    mem_summary = _dot(count[...], vvoc[...]) * 0.25
    w1 = rh_W1[...]
    rh = jnp.maximum(
        _dot_t(qemb[...][:, :H], w1[:, :H]) + _dot_t(mem_summary, w1[:, H:])
        + rh_b1r[...], 0.0)
    out[...] = _dot_t(rh, rh_W2[...]) + rh_b2r[...]


def kernel(seqs, query_tok, embed, wg_W1, wg_b1, wg_W2, wg_b2,
           eg_W1, eg_b1, eg_W2, eg_b2, rh_W1, rh_b1, rh_W2, rh_b2):
    B = seqs.shape[0]
    seqs = seqs.astype(jnp.int32)
    f32 = jnp.float32

    g_pair, v_voc = _tables(embed, wg_W1, wg_b1, wg_W2, wg_b2,
                            eg_W1, eg_b1, eg_W2, eg_b2)

    embed_pad = jnp.concatenate(
        [embed, jnp.zeros(embed.shape, jnp.float32)], axis=1)
    count, q_emb = _sc_scan_factory(B)(
        seqs.reshape(-1), query_tok.astype(jnp.int32), g_pair.reshape(-1),
        embed_pad)

    TILE = 2048
    fullg = lambda s: pl.BlockSpec(s, lambda i: tuple(0 for _ in s))
    logits = pl.pallas_call(
        _head_body,
        grid=(B // TILE,),
        out_shape=jax.ShapeDtypeStruct((B, V), f32),
        in_specs=[pl.BlockSpec((TILE, H), lambda i: (i, 0)),
                  pl.BlockSpec((TILE, 2 * H), lambda i: (i, 0)),
                  fullg((V, H)), fullg((H, 2 * H)), fullg((1, H)),
                  fullg((V, H)), fullg((1, V))],
        out_specs=pl.BlockSpec((TILE, V), lambda i: (i, 0)),
    )(count, q_emb, v_voc, rh_W1, rh_b1.reshape(1, H), rh_W2,
      rh_b2.reshape(1, V))
    return logits


# speedup vs baseline: 1.0430x; 1.0193x over previous
import functools

import jax
import jax.numpy as jnp
from jax import lax
from jax.experimental import pallas as pl
from jax.experimental.pallas import tpu as pltpu
from jax.experimental.pallas import tpu_sc as plsc

H = 64
M = 4
T = 24
V = 64
GP = 128
EMPTY = 64
NW = 32
L = 16


def _dot_t(a, b):
    return lax.dot_general(a, b, (((1,), (1,)), ((), ())),
                           preferred_element_type=jnp.float32)


def _dot(a, b):
    return lax.dot_general(a, b, (((1,), (0,)), ((), ())),
                           preferred_element_type=jnp.float32)


def _tables(embed, wg_W1, wg_b1, wg_W2, wg_b2, eg_W1, eg_b1, eg_W2, eg_b2):
    emb = embed[:V, :]
    h = jax.nn.relu(emb @ wg_W1.T + wg_b1)
    w = jax.nn.sigmoid(h @ wg_W2.T + wg_b2)
    v_voc = emb * w
    slot_vals = jnp.concatenate(
        [v_voc, jnp.zeros((1, H), jnp.float32)], axis=0)
    inp = jnp.concatenate(
        [jnp.broadcast_to(emb[:, None, :], (V, V + 1, H)),
         jnp.broadcast_to(slot_vals[None, :, :], (V, V + 1, H))],
        axis=-1)
    eh = jax.nn.relu(inp @ eg_W1.T + eg_b1)
    g = (eh @ eg_W2.T + eg_b2)[..., 0]
    g_pad = jnp.concatenate(
        [g, jnp.zeros((V, GP - (V + 1)), jnp.float32)], axis=1)
    return g_pad, v_voc


def _sc_scan_factory(B):
    rpw = B // NW
    ng = rpw // L
    mesh = plsc.VectorSubcoreMesh(core_axis_name="c", subcore_axis_name="s")
    f32 = jnp.float32

    @functools.partial(
        pl.kernel, mesh=mesh,
        out_type=(pltpu.HBM((B * H,), f32),
                  pltpu.HBM((B * H,), f32)),
        compiler_params=pltpu.CompilerParams(needs_layout_passes=False),
        scratch_types=[
            pltpu.VMEM((rpw * T,), jnp.int32),
            pltpu.VMEM((V * GP,), f32),
            pltpu.VMEM((rpw,), jnp.int32),
            pltpu.VMEM((rpw * H,), f32),
            pltpu.VMEM((rpw * H,), f32),
            pltpu.SemaphoreType.DMA,
        ],
    )
    def sc_scan(seqs_hbm, q_hbm, g_hbm, cnt_out, qoh_out,
                seqs_v, g_v, qidx_v, cnt_v, qoh_v, sem):
        wid = lax.axis_index("s") * 2 + lax.axis_index("c")
        base = wid * rpw
        pltpu.sync_copy(q_hbm.at[pl.ds(base, rpw)], qidx_v)
        pltpu.sync_copy(g_hbm, g_v)
        pltpu.sync_copy(seqs_hbm.at[pl.ds(base * T, rpw * T)], seqs_v)
        zeros16 = jnp.zeros((L,), f32)
        lane = lax.iota(jnp.int32, L)
        ones16 = jnp.ones((L,), f32)

        def group(gi, carry):
            row0 = gi * L
            zbase = row0 * H
            for j in range(L * H // L):
                cnt_v[pl.ds(zbase + j * L, L)] = zeros16
                qoh_v[pl.ds(zbase + j * L, L)] = zeros16
            empty = jnp.full((L,), EMPTY, jnp.int32)
            slots = (empty, empty, empty, empty)
            soff = gi * (L * T) + lane * T
            for t in range(T - 1):
                cur = plsc.load_gather(seqs_v, [soff + t])
                gbase = cur * GP
                logits = [plsc.load_gather(g_v, [gbase + s]) for s in slots]
                best = logits[0]
                bi = jnp.zeros((L,), jnp.int32)
                for m in range(1, M):
                    win = logits[m] > best
                    best = jnp.where(win, logits[m], best)
                    bi = jnp.where(win, jnp.full((L,), m, jnp.int32), bi)
                slots = tuple(
                    jnp.where(bi == m, cur, slots[m]) for m in range(M))
            glh = zbase + lane * H
            for m in range(M):
                plsc.addupdate_scatter(cnt_v, [glh + slots[m]], ones16,
                                       mask=slots[m] < EMPTY)
            qt = qidx_v[pl.ds(row0, L)]
            plsc.store_scatter(qoh_v, [glh + qt], ones16)
            return carry

        gpc = 128 // L
        pend = []
        for k in range(rpw // 128):
            lax.fori_loop(k * gpc, (k + 1) * gpc, group, 0)
            pend.append(pltpu.async_copy(
                cnt_v.at[pl.ds(k * 128 * H, 128 * H)],
                cnt_out.at[pl.ds((base + k * 128) * H, 128 * H)], sem))
            pend.append(pltpu.async_copy(
                qoh_v.at[pl.ds(k * 128 * H, 128 * H)],
                qoh_out.at[pl.ds((base + k * 128) * H, 128 * H)], sem))
        for c in pend:
            c.wait()

    return sc_scan


def _head_body(count, qoh, embed, vvoc, rh_W1, rh_b1r, rh_W2, rh_b2r, out):
    mem_summary = _dot(count[...], vvoc[...]) * 0.25
    q_emb = _dot(qoh[...], embed[:V, :])
    w1 = rh_W1[...]
    rh = jnp.maximum(
        _dot_t(q_emb, w1[:, :H]) + _dot_t(mem_summary, w1[:, H:])
        + rh_b1r[...], 0.0)
    out[...] = _dot_t(rh, rh_W2[...]) + rh_b2r[...]


def kernel(seqs, query_tok, embed, wg_W1, wg_b1, wg_W2, wg_b2,
           eg_W1, eg_b1, eg_W2, eg_b2, rh_W1, rh_b1, rh_W2, rh_b2):
    B = seqs.shape[0]
    seqs = seqs.astype(jnp.int32)
    f32 = jnp.float32

    g_pair, v_voc = _tables(embed, wg_W1, wg_b1, wg_W2, wg_b2,
                            eg_W1, eg_b1, eg_W2, eg_b2)

    count, q_oh = _sc_scan_factory(B)(
        seqs.reshape(-1), query_tok.astype(jnp.int32), g_pair.reshape(-1))
    count = count.reshape(B, H)
    q_oh = q_oh.reshape(B, H)

    TILE = 2048
    fullg = lambda s: pl.BlockSpec(s, lambda i: tuple(0 for _ in s))
    logits = pl.pallas_call(
        _head_body,
        grid=(B // TILE,),
        out_shape=jax.ShapeDtypeStruct((B, V), f32),
        in_specs=[pl.BlockSpec((TILE, H), lambda i: (i, 0)),
                  pl.BlockSpec((TILE, H), lambda i: (i, 0)),
                  fullg((embed.shape[0], H)), fullg((V, H)),
                  fullg((H, 2 * H)), fullg((1, H)),
                  fullg((V, H)), fullg((1, V))],
        out_specs=pl.BlockSpec((TILE, V), lambda i: (i, 0)),
    )(count, q_oh, embed, v_voc, rh_W1, rh_b1.reshape(1, H), rh_W2,
      rh_b2.reshape(1, V))
    return logits


# speedup vs baseline: 1.2953x; 1.2420x over previous
import functools

import jax
import jax.numpy as jnp
from jax import lax
from jax.experimental import pallas as pl
from jax.experimental.pallas import tpu as pltpu
from jax.experimental.pallas import tpu_sc as plsc

H = 64
M = 4
T = 24
V = 64
GP = 128
EMPTY = 64
NW = 32
L = 16


def _dot_t(a, b):
    return lax.dot_general(a, b, (((1,), (1,)), ((), ())),
                           preferred_element_type=jnp.float32)


def _dot(a, b):
    return lax.dot_general(a, b, (((1,), (0,)), ((), ())),
                           preferred_element_type=jnp.float32)


def _tables(embed, wg_W1, wg_b1, wg_W2, wg_b2, eg_W1, eg_b1, eg_W2, eg_b2):
    emb = embed[:V, :]
    h = jax.nn.relu(emb @ wg_W1.T + wg_b1)
    w = jax.nn.sigmoid(h @ wg_W2.T + wg_b2)
    v_voc = emb * w
    slot_vals = jnp.concatenate(
        [v_voc, jnp.zeros((1, H), jnp.float32)], axis=0)
    inp = jnp.concatenate(
        [jnp.broadcast_to(emb[:, None, :], (V, V + 1, H)),
         jnp.broadcast_to(slot_vals[None, :, :], (V, V + 1, H))],
        axis=-1)
    eh = jax.nn.relu(inp @ eg_W1.T + eg_b1)
    g = (eh @ eg_W2.T + eg_b2)[..., 0]
    g_pad = jnp.concatenate(
        [g, jnp.zeros((V, GP - (V + 1)), jnp.float32)], axis=1)
    return g_pad, v_voc


def _head_tables(embed, v_voc, rh_W1):
    ch = v_voc @ rh_W1[:, H:].T * 0.25
    qh = embed[:V, :] @ rh_W1[:, :H].T
    return jnp.concatenate([ch, qh], axis=0)


def _sc_scan_factory(B):
    rpw = B // NW
    ng = rpw // L
    mesh = plsc.VectorSubcoreMesh(core_axis_name="c", subcore_axis_name="s")
    f32 = jnp.float32

    @functools.partial(
        pl.kernel, mesh=mesh,
        out_type=pltpu.HBM((B * 2 * H,), f32),
        compiler_params=pltpu.CompilerParams(needs_layout_passes=False),
        scratch_types=[
            pltpu.VMEM((rpw * T,), jnp.int32),
            pltpu.VMEM((V * GP,), f32),
            pltpu.VMEM((rpw,), jnp.int32),
            pltpu.VMEM((rpw * 2 * H,), f32),
            pltpu.SemaphoreType.DMA,
        ],
    )
    def sc_scan(seqs_hbm, q_hbm, g_hbm, dat_out,
                seqs_v, g_v, qidx_v, cnt_v, sem):
        wid = lax.axis_index("s") * 2 + lax.axis_index("c")
        base = wid * rpw
        c1 = pltpu.async_copy(q_hbm.at[pl.ds(base, rpw)], qidx_v, sem)
        c2 = pltpu.async_copy(g_hbm, g_v, sem)
        c3 = pltpu.async_copy(
            seqs_hbm.at[pl.ds(base * T, rpw * T)], seqs_v, sem)
        c1.wait(); c2.wait(); c3.wait()
        zeros16 = jnp.zeros((L,), f32)
        lane = lax.iota(jnp.int32, L)
        ones16 = jnp.ones((L,), f32)

        def group(gi, carry):
            row0 = gi * L
            zbase = row0 * 2 * H
            for j in range(2 * L * H // L):
                cnt_v[pl.ds(zbase + j * L, L)] = zeros16
            empty = jnp.full((L,), EMPTY, jnp.int32)
            slots = (empty, empty, empty, empty)
            soff = gi * (L * T) + lane * T
            for t in range(T - 1):
                cur = plsc.load_gather(seqs_v, [soff + t])
                gbase = cur * GP
                logits = [plsc.load_gather(g_v, [gbase + s]) for s in slots]
                best = logits[0]
                bi = jnp.zeros((L,), jnp.int32)
                for m in range(1, M):
                    win = logits[m] > best
                    best = jnp.where(win, logits[m], best)
                    bi = jnp.where(win, jnp.full((L,), m, jnp.int32), bi)
                slots = tuple(
                    jnp.where(bi == m, cur, slots[m]) for m in range(M))
            glh = zbase + lane * (2 * H)
            for m in range(M):
                plsc.addupdate_scatter(cnt_v, [glh + slots[m]], ones16,
                                       mask=slots[m] < EMPTY)
            qt = qidx_v[pl.ds(row0, L)]
            plsc.store_scatter(cnt_v, [glh + H + qt], ones16)
            return carry

        gpc = 128 // L
        pend = []
        for k in range(rpw // 128):
            lax.fori_loop(k * gpc, (k + 1) * gpc, group, 0)
            pend.append(pltpu.async_copy(
                cnt_v.at[pl.ds(k * 256 * H, 256 * H)],
                dat_out.at[pl.ds((base + k * 128) * 2 * H, 256 * H)], sem))
        for c in pend:
            c.wait()

    return sc_scan


def _head_body(data, wc, rh_b1r, rh_W2, rh_b2r, out):
    rh = jnp.maximum(_dot(data[...], wc[...]) + rh_b1r[...], 0.0)
    out[...] = _dot_t(rh, rh_W2[...]) + rh_b2r[...]


def kernel(seqs, query_tok, embed, wg_W1, wg_b1, wg_W2, wg_b2,
           eg_W1, eg_b1, eg_W2, eg_b2, rh_W1, rh_b1, rh_W2, rh_b2):
    B = seqs.shape[0]
    seqs = seqs.astype(jnp.int32)
    f32 = jnp.float32

    g_pair, v_voc = _tables(embed, wg_W1, wg_b1, wg_W2, wg_b2,
                            eg_W1, eg_b1, eg_W2, eg_b2)

    wc = _head_tables(embed, v_voc, rh_W1)
    data = _sc_scan_factory(B)(
        seqs.reshape(-1), query_tok.astype(jnp.int32),
        g_pair.reshape(-1)).reshape(B, 2 * H)

    TILE = 4096
    fullg = lambda s: pl.BlockSpec(s, lambda i: tuple(0 for _ in s))
    logits = pl.pallas_call(
        _head_body,
        grid=(B // TILE,),
        out_shape=jax.ShapeDtypeStruct((B, V), f32),
        in_specs=[pl.BlockSpec((TILE, 2 * H), lambda i: (i, 0)),
                  fullg((2 * H, H)), fullg((1, H)),
                  fullg((V, H)), fullg((1, V))],
        out_specs=pl.BlockSpec((TILE, V), lambda i: (i, 0)),
    )(data, wc, rh_b1.reshape(1, H), rh_W2, rh_b2.reshape(1, V))
    return logits


# speedup vs baseline: 1.3405x; 1.0349x over previous
import functools

import jax
import jax.numpy as jnp
from jax import lax
from jax.experimental import pallas as pl
from jax.experimental.pallas import tpu as pltpu
from jax.experimental.pallas import tpu_sc as plsc

H = 64
M = 4
T = 24
V = 64
GP = 128
EMPTY = 64
NW = 32
L = 16


def _dot_t(a, b):
    return lax.dot_general(a, b, (((1,), (1,)), ((), ())),
                           preferred_element_type=jnp.float32)


def _dot(a, b):
    return lax.dot_general(a, b, (((1,), (0,)), ((), ())),
                           preferred_element_type=jnp.float32)


def _tables(embed, wg_W1, wg_b1, wg_W2, wg_b2, eg_W1, eg_b1, eg_W2, eg_b2):
    emb = embed[:V, :]
    h = jax.nn.relu(emb @ wg_W1.T + wg_b1)
    w = jax.nn.sigmoid(h @ wg_W2.T + wg_b2)
    v_voc = emb * w
    slot_vals = jnp.concatenate(
        [v_voc, jnp.zeros((1, H), jnp.float32)], axis=0)
    inp = jnp.concatenate(
        [jnp.broadcast_to(emb[:, None, :], (V, V + 1, H)),
         jnp.broadcast_to(slot_vals[None, :, :], (V, V + 1, H))],
        axis=-1)
    eh = jax.nn.relu(inp @ eg_W1.T + eg_b1)
    g = (eh @ eg_W2.T + eg_b2)[..., 0]
    return g, v_voc


def _head_tables(embed, v_voc, rh_W1):
    ch = v_voc @ rh_W1[:, H:].T * 0.25
    qh = embed[:V, :] @ rh_W1[:, :H].T
    return jnp.concatenate([ch, qh], axis=0)


def _sc_scan_factory(B):
    rpw = B // NW
    ng = rpw // L
    mesh = plsc.VectorSubcoreMesh(core_axis_name="c", subcore_axis_name="s")
    f32 = jnp.float32

    @functools.partial(
        pl.kernel, mesh=mesh,
        out_type=pltpu.HBM((B * 2 * H,), f32),
        compiler_params=pltpu.CompilerParams(needs_layout_passes=False),
        scratch_types=[
            pltpu.VMEM((rpw * T,), jnp.int32),
            pltpu.VMEM((V * (V + 1),), f32),
            pltpu.VMEM((rpw,), jnp.int32),
            pltpu.VMEM((rpw * 2 * H,), f32),
            pltpu.SemaphoreType.DMA,
        ],
    )
    def sc_scan(seqs_hbm, q_hbm, g_hbm, dat_out,
                seqs_v, g_v, qidx_v, cnt_v, sem):
        wid = lax.axis_index("s") * 2 + lax.axis_index("c")
        base = wid * rpw
        c1 = pltpu.async_copy(q_hbm.at[pl.ds(base, rpw)], qidx_v, sem)
        c2 = pltpu.async_copy(g_hbm, g_v, sem)
        c3 = pltpu.async_copy(
            seqs_hbm.at[pl.ds(base * T, rpw * T)], seqs_v, sem)
        c1.wait(); c2.wait(); c3.wait()
        zeros16 = jnp.zeros((L,), f32)
        lane = lax.iota(jnp.int32, L)
        ones16 = jnp.ones((L,), f32)

        def group(gi, carry):
            row0 = gi * L
            zbase = row0 * 2 * H
            for j in range(2 * L * H // L):
                cnt_v[pl.ds(zbase + j * L, L)] = zeros16
            empty = jnp.full((L,), EMPTY, jnp.int32)
            soff = gi * (L * T) + lane * T
            cur = plsc.load_gather(seqs_v, [soff])
            slots = (cur, empty, empty, empty)
            cur = plsc.load_gather(seqs_v, [soff + 1])
            gbase = cur * (V + 1)
            l0 = plsc.load_gather(g_v, [gbase + slots[0]])
            le = plsc.load_gather(g_v, [gbase + empty])
            win1 = le > l0
            slots = (jnp.where(win1, slots[0], cur),
                     jnp.where(win1, cur, empty), empty, empty)
            for t in range(2, T - 1):
                cur = plsc.load_gather(seqs_v, [soff + t])
                gbase = cur * (V + 1)
                logits = [plsc.load_gather(g_v, [gbase + s]) for s in slots]
                best = logits[0]
                bi = jnp.zeros((L,), jnp.int32)
                for m in range(1, M):
                    win = logits[m] > best
                    best = jnp.where(win, logits[m], best)
                    bi = jnp.where(win, jnp.full((L,), m, jnp.int32), bi)
                slots = tuple(
                    jnp.where(bi == m, cur, slots[m]) for m in range(M))
            glh = zbase + lane * (2 * H)
            for m in range(M):
                plsc.addupdate_scatter(cnt_v, [glh + slots[m]], ones16,
                                       mask=slots[m] < EMPTY)
            qt = qidx_v[pl.ds(row0, L)]
            plsc.store_scatter(cnt_v, [glh + H + qt], ones16)
            return carry

        gpc = 128 // L
        pend = []
        for k in range(rpw // 128):
            lax.fori_loop(k * gpc, (k + 1) * gpc, group, 0)
            pend.append(pltpu.async_copy(
                cnt_v.at[pl.ds(k * 256 * H, 256 * H)],
                dat_out.at[pl.ds((base + k * 128) * 2 * H, 256 * H)], sem))
        for c in pend:
            c.wait()

    return sc_scan


def _head_body(data, wc, rh_b1r, rh_W2, rh_b2r, out):
    rh = jnp.maximum(_dot(data[...], wc[...]) + rh_b1r[...], 0.0)
    out[...] = _dot_t(rh, rh_W2[...]) + rh_b2r[...]


def kernel(seqs, query_tok, embed, wg_W1, wg_b1, wg_W2, wg_b2,
           eg_W1, eg_b1, eg_W2, eg_b2, rh_W1, rh_b1, rh_W2, rh_b2):
    B = seqs.shape[0]
    seqs = seqs.astype(jnp.int32)
    f32 = jnp.float32

    g_pair, v_voc = _tables(embed, wg_W1, wg_b1, wg_W2, wg_b2,
                            eg_W1, eg_b1, eg_W2, eg_b2)

    wc = _head_tables(embed, v_voc, rh_W1)
    data = _sc_scan_factory(B)(
        seqs.reshape(-1), query_tok.astype(jnp.int32),
        g_pair.reshape(-1)).reshape(B, 2 * H)

    TILE = 4096
    fullg = lambda s: pl.BlockSpec(s, lambda i: tuple(0 for _ in s))
    logits = pl.pallas_call(
        _head_body,
        grid=(B // TILE,),
        out_shape=jax.ShapeDtypeStruct((B, V), f32),
        in_specs=[pl.BlockSpec((TILE, 2 * H), lambda i: (i, 0)),
                  fullg((2 * H, H)), fullg((1, H)),
                  fullg((V, H)), fullg((1, V))],
        out_specs=pl.BlockSpec((TILE, V), lambda i: (i, 0)),
    )(data, wc, rh_b1.reshape(1, H), rh_W2, rh_b2.reshape(1, V))
    return logits


# speedup vs baseline: 1.3417x; 1.0009x over previous
import functools

import jax
import jax.numpy as jnp
from jax import lax
from jax.experimental import pallas as pl
from jax.experimental.pallas import tpu as pltpu
from jax.experimental.pallas import tpu_sc as plsc

H = 64
M = 4
T = 24
V = 64
EMPTY = 64
NW = 32
L = 16


def _dot_t(a, b):
    return lax.dot_general(a, b, (((1,), (1,)), ((), ())),
                           preferred_element_type=jnp.float32)


def _dot(a, b):
    return lax.dot_general(a, b, (((1,), (0,)), ((), ())),
                           preferred_element_type=jnp.float32)


def _tables(embed, wg_W1, wg_b1, wg_W2, wg_b2, eg_W1, eg_b1, eg_W2, eg_b2):
    emb = embed[:V, :]
    h = jax.nn.relu(emb @ wg_W1.T + wg_b1)
    w = jax.nn.sigmoid(h @ wg_W2.T + wg_b2)
    v_voc = emb * w
    slot_vals = jnp.concatenate(
        [v_voc, jnp.zeros((1, H), jnp.float32)], axis=0)
    inp = jnp.concatenate(
        [jnp.broadcast_to(emb[:, None, :], (V, V + 1, H)),
         jnp.broadcast_to(slot_vals[None, :, :], (V, V + 1, H))],
        axis=-1)
    eh = jax.nn.relu(inp @ eg_W1.T + eg_b1)
    g = (eh @ eg_W2.T + eg_b2)[..., 0]
    return g, v_voc


def _head_tables(embed, v_voc, rh_W1):
    ch = v_voc @ rh_W1[:, H:].T * 0.25
    qh = embed[:V, :] @ rh_W1[:, :H].T
    return jnp.concatenate([ch, qh], axis=0)


def _sc_scan_factory(B):
    rpw = B // NW
    ng = rpw // L
    mesh = plsc.VectorSubcoreMesh(core_axis_name="c", subcore_axis_name="s")
    f32 = jnp.float32

    @functools.partial(
        pl.kernel, mesh=mesh,
        out_type=pltpu.HBM((B * 2 * H,), f32),
        compiler_params=pltpu.CompilerParams(needs_layout_passes=False),
        scratch_types=[
            pltpu.VMEM((rpw * T,), jnp.int32),
            pltpu.VMEM((V * (V + 1),), f32),
            pltpu.VMEM((rpw,), jnp.int32),
            pltpu.VMEM((rpw * 2 * H,), f32),
            pltpu.SemaphoreType.DMA,
        ],
    )
    def sc_scan(seqs_hbm, q_hbm, g_hbm, dat_out,
                seqs_v, g_v, qidx_v, cnt_v, sem):
        wid = lax.axis_index("s") * 2 + lax.axis_index("c")
        base = wid * rpw
        c1 = pltpu.async_copy(q_hbm.at[pl.ds(base, rpw)], qidx_v, sem)
        c2 = pltpu.async_copy(g_hbm, g_v, sem)
        c3 = pltpu.async_copy(
            seqs_hbm.at[pl.ds(base * T, rpw * T)], seqs_v, sem)
        c1.wait(); c2.wait(); c3.wait()
        zeros16 = jnp.zeros((L,), f32)
        lane = lax.iota(jnp.int32, L)
        ones16 = jnp.ones((L,), f32)

        def group(gi, carry):
            row0 = gi * L
            zbase = row0 * 2 * H
            for j in range(2 * L * H // L):
                cnt_v[pl.ds(zbase + j * L, L)] = zeros16
            empty = jnp.full((L,), EMPTY, jnp.int32)
            soff = gi * (L * T) + lane * T
            cur = plsc.load_gather(seqs_v, [soff])
            slots = (cur, empty, empty, empty)
            cur = plsc.load_gather(seqs_v, [soff + 1])
            gbase = cur * (V + 1)
            l0 = plsc.load_gather(g_v, [gbase + slots[0]])
            le = plsc.load_gather(g_v, [gbase + empty])
            win1 = le > l0
            slots = (jnp.where(win1, slots[0], cur),
                     jnp.where(win1, cur, empty), empty, empty)
            for t in range(2, T - 1):
                cur = plsc.load_gather(seqs_v, [soff + t])
                gbase = cur * (V + 1)
                logits = [plsc.load_gather(g_v, [gbase + s]) for s in slots]
                best = logits[0]
                bi = jnp.zeros((L,), jnp.int32)
                for m in range(1, M):
                    win = logits[m] > best
                    best = jnp.where(win, logits[m], best)
                    bi = jnp.where(win, jnp.full((L,), m, jnp.int32), bi)
                slots = tuple(
                    jnp.where(bi == m, cur, slots[m]) for m in range(M))
            glh = zbase + lane * (2 * H)
            for m in range(M):
                plsc.addupdate_scatter(cnt_v, [glh + slots[m]], ones16,
                                       mask=slots[m] < EMPTY)
            qt = qidx_v[pl.ds(row0, L)]
            plsc.store_scatter(cnt_v, [glh + H + qt], ones16)
            return carry

        gpc = 128 // L
        pend = []
        for k in range(rpw // 128):
            lax.fori_loop(k * gpc, (k + 1) * gpc, group, 0)
            pend.append(pltpu.async_copy(
                cnt_v.at[pl.ds(k * 256 * H, 256 * H)],
                dat_out.at[pl.ds((base + k * 128) * 2 * H, 256 * H)], sem))
        for c in pend:
            c.wait()

    return sc_scan


def _head_body(data, wc, rh_b1r, rh_W2, rh_b2r, out):
    rh = jnp.maximum(_dot(data[...], wc[...]) + rh_b1r[...], 0.0)
    out[...] = _dot_t(rh, rh_W2[...]) + rh_b2r[...]


def kernel(seqs, query_tok, embed, wg_W1, wg_b1, wg_W2, wg_b2,
           eg_W1, eg_b1, eg_W2, eg_b2, rh_W1, rh_b1, rh_W2, rh_b2):
    B = seqs.shape[0]
    seqs = seqs.astype(jnp.int32)
    f32 = jnp.float32

    g_pair, v_voc = _tables(embed, wg_W1, wg_b1, wg_W2, wg_b2,
                            eg_W1, eg_b1, eg_W2, eg_b2)

    wc = _head_tables(embed, v_voc, rh_W1)
    data = _sc_scan_factory(B)(
        seqs.reshape(-1), query_tok.astype(jnp.int32),
        g_pair.reshape(-1)).reshape(B, 2 * H)

    TILE = 4096
    fullg = lambda s: pl.BlockSpec(s, lambda i: tuple(0 for _ in s))
    logits = pl.pallas_call(
        _head_body,
        grid=(B // TILE,),
        out_shape=jax.ShapeDtypeStruct((B, V), f32),
        in_specs=[pl.BlockSpec((TILE, 2 * H), lambda i: (i, 0)),
                  fullg((2 * H, H)), fullg((1, H)),
                  fullg((V, H)), fullg((1, V))],
        out_specs=pl.BlockSpec((TILE, V), lambda i: (i, 0)),
    )(data, wc, rh_b1.reshape(1, H), rh_W2, rh_b2.reshape(1, V))
    return logits
